# Initial kernel scaffold; baseline (speedup 1.0000x reference)
import jax
import jax.numpy as jnp
from jax.experimental import pallas as pl


def kernel(inputs, embs_weight):
    raise NotImplementedError("write your pallas kernel here")


# speedup vs baseline: 1.1790x; 1.1790x over previous
import functools

import jax
import jax.numpy as jnp
from jax import lax
from jax.experimental import pallas as pl
from jax.experimental.pallas import tpu as pltpu
from jax.experimental.pallas import tpu_sc as plsc

_N_SPECIAL = 8
_PAD_IDX = _N_SPECIAL

_NC = 2
_NS = 16
_NW = _NC * _NS
_L = 16


@functools.lru_cache(maxsize=None)
def _build(n_rows: int, d: int, n_vocab: int):
    dp = -(-d // _L) * _L
    rpb = 64
    assert n_rows % (_NW * rpb * 2) == 0
    b_per_w = n_rows // _NW
    n_iter = b_per_w // (rpb * 2)
    pb = rpb * d
    k_full, rem = divmod(d, _L)
    mesh = plsc.VectorSubcoreMesh(core_axis_name="c", subcore_axis_name="s")

    @functools.partial(
        pl.kernel,
        mesh=mesh,
        out_type=jax.ShapeDtypeStruct((n_rows * d,), jnp.float32),
        scratch_types=[
            pltpu.VMEM((b_per_w,), jnp.int32),
            pltpu.VMEM((n_vocab * dp,), jnp.float32),
            pltpu.VMEM((2 * pb,), jnp.float32),
            pltpu.SemaphoreType.DMA,
            pltpu.SemaphoreType.DMA,
        ],
        compiler_params=pltpu.CompilerParams(
            use_tc_tiling_on_sc=False, needs_layout_passes=False),
    )
    def emb(idx_hbm, tab_hbm, out_hbm, idx_v, tab_v, pack_v, osem0, osem1):
        wid = lax.axis_index("s") * _NC + lax.axis_index("c")
        base = wid * b_per_w
        pltpu.sync_copy(idx_hbm.at[pl.ds(base, b_per_w)], idx_v)
        pltpu.sync_copy(tab_hbm, tab_v)

        zeros = jnp.zeros((_L,), jnp.float32)
        for r in (0, _PAD_IDX):
            for k in range(dp // _L):
                tab_v[pl.ds(r * dp + k * _L, _L)] = zeros

        iota = lax.iota(jnp.int32, _L)
        tail_mask = iota < rem
        osems = (osem0, osem1)

        def outer(gg, carry):
            for b in range(2):
                gbuf = gg * 2 + b

                @pl.when(gg > 0)
                def _drain():
                    pltpu.make_async_copy(
                        pack_v.at[pl.ds(b * pb, pb)],
                        out_hbm.at[pl.ds(0, pb)],
                        osems[b],
                    ).wait()

                def row(jr, c2):
                    j = gbuf * rpb + jr
                    vidx = plsc.load_gather(idx_v, [jnp.full((_L,), j, jnp.int32)])
                    srcb = vidx * dp + iota
                    dstb = jnp.full((_L,), b * pb + jr * d, jnp.int32) + iota
                    for k in range(k_full):
                        v = plsc.load_gather(tab_v, [srcb + k * _L])
                        plsc.store_scatter(pack_v, [dstb + k * _L], v)
                    v = plsc.load_gather(tab_v, [srcb + k_full * _L])
                    plsc.store_scatter(pack_v, [dstb + k_full * _L], v,
                                       mask=tail_mask)
                    return c2

                lax.fori_loop(0, rpb, row, 0)
                pltpu.async_copy(
                    pack_v.at[pl.ds(b * pb, pb)],
                    out_hbm.at[pl.ds(base * d + gbuf * pb, pb)],
                    osems[b],
                )
            return carry

        lax.fori_loop(0, n_iter, outer, 0)
        for b in range(2):
            pltpu.make_async_copy(
                pack_v.at[pl.ds(b * pb, pb)],
                out_hbm.at[pl.ds(0, pb)],
                osems[b],
            ).wait()

    return emb


def kernel(inputs, embs_weight):
    b, l = inputs.shape
    n_vocab, d = embs_weight.shape
    n_rows = b * l
    dp = -(-d // _L) * _L
    tab_flat = jnp.pad(embs_weight, ((0, 0), (0, dp - d))).reshape(-1)
    out = _build(n_rows, d, n_vocab)(inputs.reshape(-1), tab_flat)
    return out.reshape(b, l, d)


# speedup vs baseline: 1.4226x; 1.2066x over previous
import functools

import jax
import jax.numpy as jnp
from jax import lax
from jax.experimental import pallas as pl
from jax.experimental.pallas import tpu as pltpu
from jax.experimental.pallas import tpu_sc as plsc

_N_SPECIAL = 8
_PAD_IDX = _N_SPECIAL

_NC = 2
_NS = 16
_NW = _NC * _NS
_L = 16


@functools.lru_cache(maxsize=None)
def _build(bsz: int, seq: int, d: int, n_vocab: int):
    dp = -(-d // _L) * _L
    assert bsz % (_NW * 2) == 0
    b_per_w = bsz // _NW
    n_iter = b_per_w // 2
    pb = seq * d
    k_full, rem = divmod(d, _L)
    mesh = plsc.VectorSubcoreMesh(core_axis_name="c", subcore_axis_name="s")

    @functools.partial(
        pl.kernel,
        mesh=mesh,
        out_type=jax.ShapeDtypeStruct((bsz, seq, d), jnp.float32),
        scratch_types=[
            pltpu.VMEM((b_per_w * seq,), jnp.int32),
            pltpu.VMEM((n_vocab * dp,), jnp.float32),
            pltpu.VMEM((2 * seq, d), jnp.float32),
            pltpu.SemaphoreType.DMA,
            pltpu.SemaphoreType.DMA,
        ],
        compiler_params=pltpu.CompilerParams(
            use_tc_tiling_on_sc=False, needs_layout_passes=False),
    )
    def emb(idx_hbm, tab_hbm, out_hbm, idx_v, tab_v, pack_v, osem0, osem1):
        wid = lax.axis_index("s") * _NC + lax.axis_index("c")
        base = wid * b_per_w
        pltpu.sync_copy(idx_hbm.at[pl.ds(base * seq, b_per_w * seq)], idx_v)
        pltpu.sync_copy(tab_hbm, tab_v)

        zeros = jnp.zeros((_L,), jnp.float32)
        for r in (0, _PAD_IDX):
            for k in range(dp // _L):
                tab_v[pl.ds(r * dp + k * _L, _L)] = zeros

        iota = lax.iota(jnp.int32, _L)
        tail_mask = iota < rem
        osems = (osem0, osem1)

        def outer(gg, carry):
            for b in range(2):
                gbuf = gg * 2 + b

                @pl.when(gg > 0)
                def _drain():
                    pltpu.make_async_copy(
                        pack_v.at[pl.ds(b * seq, seq)],
                        out_hbm.at[0],
                        osems[b],
                    ).wait()

                def row(jr, c2):
                    j = gbuf * seq + jr
                    vidx = plsc.load_gather(idx_v, [jnp.full((_L,), j, jnp.int32)])
                    srcb = vidx * dp + iota
                    vrow = jnp.full((_L,), b * seq + jr, jnp.int32)
                    for k in range(k_full):
                        v = plsc.load_gather(tab_v, [srcb + k * _L])
                        plsc.store_scatter(pack_v, [vrow, iota + k * _L], v)
                    v = plsc.load_gather(tab_v, [srcb + k_full * _L])
                    plsc.store_scatter(pack_v, [vrow, iota + k_full * _L], v,
                                       mask=tail_mask)
                    return c2

                lax.fori_loop(0, seq, row, 0)
                pltpu.async_copy(
                    pack_v.at[pl.ds(b * seq, seq)],
                    out_hbm.at[base + gbuf],
                    osems[b],
                )
            return carry

        lax.fori_loop(0, n_iter, outer, 0)
        for b in range(2):
            pltpu.make_async_copy(
                pack_v.at[pl.ds(b * seq, seq)],
                out_hbm.at[0],
                osems[b],
            ).wait()

    return emb


def kernel(inputs, embs_weight):
    bsz, seq = inputs.shape
    n_vocab, d = embs_weight.shape
    dp = -(-d // _L) * _L
    tab_flat = jnp.pad(embs_weight, ((0, 0), (0, dp - d))).reshape(-1)
    return _build(bsz, seq, d, n_vocab)(inputs.reshape(-1), tab_flat)


# speedup vs baseline: 1.7929x; 1.2603x over previous
import functools

import jax
import jax.numpy as jnp
from jax import lax
from jax.experimental import pallas as pl
from jax.experimental.pallas import tpu as pltpu
from jax.experimental.pallas import tpu_sc as plsc

_N_SPECIAL = 8
_PAD_IDX = _N_SPECIAL

_NC = 2
_NS = 16
_NW = _NC * _NS
_L = 16
_NB = 2


@functools.lru_cache(maxsize=None)
def _build(bsz: int, seq: int, d: int, n_vocab: int):
    dp = -(-d // _L) * _L
    nb = _NB
    assert bsz % (_NW * 2 * nb) == 0
    b_per_w = bsz // _NW
    n_iter = b_per_w // (2 * nb)
    k_full, rem = divmod(d, _L)
    mesh = plsc.VectorSubcoreMesh(core_axis_name="c", subcore_axis_name="s")

    @functools.partial(
        pl.kernel,
        mesh=mesh,
        out_type=jax.ShapeDtypeStruct((bsz, seq, d), jnp.float32),
        scratch_types=[
            pltpu.VMEM((b_per_w * seq,), jnp.int32),
            pltpu.VMEM((n_vocab * dp,), jnp.float32),
            pltpu.VMEM((2, nb, seq, d), jnp.float32),
            pltpu.SemaphoreType.DMA,
            pltpu.SemaphoreType.DMA,
        ],
        compiler_params=pltpu.CompilerParams(
            use_tc_tiling_on_sc=True, needs_layout_passes=False),
    )
    def emb(idx_hbm, tab_hbm, out_hbm, idx_v, tab_v, pack_v, osem0, osem1):
        wid = lax.axis_index("s") * _NC + lax.axis_index("c")
        base = wid * b_per_w
        pltpu.sync_copy(idx_hbm.at[pl.ds(base * seq, b_per_w * seq)], idx_v)
        pltpu.sync_copy(tab_hbm, tab_v)

        zeros = jnp.zeros((_L,), jnp.float32)
        for r in (0, _PAD_IDX):
            for k in range(dp // _L):
                tab_v[pl.ds(r * dp + k * _L, _L)] = zeros

        iota = lax.iota(jnp.int32, _L)
        tail_mask = iota < rem
        osems = (osem0, osem1)

        def outer(gg, carry):
            for b in range(2):
                slab = gg * 2 + b
                bi0 = slab * nb

                @pl.when(gg > 0)
                def _drain():
                    pltpu.make_async_copy(
                        pack_v.at[b],
                        out_hbm.at[pl.ds(0, nb)],
                        osems[b],
                    ).wait()

                vb = jnp.full((_L,), b, jnp.int32)
                for nbi in range(nb):
                    vnb = jnp.full((_L,), nbi, jnp.int32)

                    def row(jr, c2):
                        j = (bi0 + nbi) * seq + jr
                        vidx = plsc.load_gather(
                            idx_v, [jnp.full((_L,), j, jnp.int32)])
                        srcb = vidx * dp + iota
                        vrow = jnp.full((_L,), jr, jnp.int32)
                        for k in range(k_full):
                            v = plsc.load_gather(tab_v, [srcb + k * _L])
                            plsc.store_scatter(
                                pack_v, [vb, vnb, vrow, iota + k * _L], v)
                        v = plsc.load_gather(tab_v, [srcb + k_full * _L])
                        plsc.store_scatter(
                            pack_v, [vb, vnb, vrow, iota + k_full * _L], v,
                            mask=tail_mask)
                        return c2

                    lax.fori_loop(0, seq, row, 0)

                pltpu.async_copy(
                    pack_v.at[b],
                    out_hbm.at[pl.ds(base + bi0, nb)],
                    osems[b],
                )
            return carry

        lax.fori_loop(0, n_iter, outer, 0)
        for b in range(2):
            pltpu.make_async_copy(
                pack_v.at[b],
                out_hbm.at[pl.ds(0, nb)],
                osems[b],
            ).wait()

    return emb


def kernel(inputs, embs_weight):
    bsz, seq = inputs.shape
    n_vocab, d = embs_weight.shape
    dp = -(-d // _L) * _L
    tab_flat = jnp.pad(embs_weight, ((0, 0), (0, dp - d))).reshape(-1)
    return _build(bsz, seq, d, n_vocab)(inputs.reshape(-1), tab_flat)


# speedup vs baseline: 2.5979x; 1.4490x over previous
import functools

import jax
import jax.numpy as jnp
from jax import lax
from jax.experimental import pallas as pl
from jax.experimental.pallas import tpu as pltpu
from jax.experimental.pallas import tpu_sc as plsc

_N_SPECIAL = 8
_PAD_IDX = _N_SPECIAL

_NC = 2
_NS = 16
_NW = _NC * _NS
_L = 16
_NB = 2


@functools.lru_cache(maxsize=None)
def _build(bsz: int, seq: int, d: int, n_vocab: int):
    dp = -(-d // _L) * _L
    nb = _NB
    assert bsz % (_NW * 2 * nb) == 0
    b_per_w = bsz // _NW
    n_iter = b_per_w // (2 * nb)
    k_full, rem = divmod(d, _L)
    mesh = plsc.VectorSubcoreMesh(core_axis_name="c", subcore_axis_name="s")

    @functools.partial(
        pl.kernel,
        mesh=mesh,
        out_type=jax.ShapeDtypeStruct((bsz, seq, d), jnp.float32),
        scratch_types=[
            pltpu.VMEM((b_per_w * seq,), jnp.int32),
            pltpu.VMEM((n_vocab * dp,), jnp.float32),
            pltpu.VMEM((2, nb, seq, d), jnp.float32),
            pltpu.SemaphoreType.DMA,
            pltpu.SemaphoreType.DMA,
        ],
        compiler_params=pltpu.CompilerParams(
            use_tc_tiling_on_sc=True, needs_layout_passes=False),
    )
    def emb(idx_hbm, tab_hbm, out_hbm, idx_v, tab_v, pack_v, osem0, osem1):
        wid = lax.axis_index("s") * _NC + lax.axis_index("c")
        base = wid * b_per_w
        pltpu.sync_copy(idx_hbm.at[pl.ds(base * seq, b_per_w * seq)], idx_v)
        pltpu.sync_copy(tab_hbm, tab_v)

        zeros = jnp.zeros((_L,), jnp.float32)
        for r in (0, _PAD_IDX):
            for k in range(dp // _L):
                tab_v[pl.ds(r * dp + k * _L, _L)] = zeros

        iota = lax.iota(jnp.int32, _L)
        tail_mask = iota < rem
        osems = (osem0, osem1)

        def outer(gg, carry):
            for b in range(2):
                slab = gg * 2 + b
                bi0 = slab * nb

                @pl.when(gg > 0)
                def _drain():
                    pltpu.make_async_copy(
                        pack_v.at[b],
                        out_hbm.at[pl.ds(0, nb)],
                        osems[b],
                    ).wait()

                vb = jnp.full((_L,), b, jnp.int32)
                for nbi in range(nb):
                    vnb = jnp.full((_L,), nbi, jnp.int32)

                    @plsc.parallel_loop(0, seq, unroll=4)
                    def _row(jr):
                        j = (bi0 + nbi) * seq + jr
                        vidx = plsc.load_gather(
                            idx_v, [jnp.full((_L,), j, jnp.int32)])
                        srcb = vidx * dp + iota
                        vrow = jnp.full((_L,), jr, jnp.int32)
                        for k in range(k_full):
                            v = plsc.load_gather(tab_v, [srcb + k * _L])
                            plsc.store_scatter(
                                pack_v, [vb, vnb, vrow, iota + k * _L], v)
                        v = plsc.load_gather(tab_v, [srcb + k_full * _L])
                        plsc.store_scatter(
                            pack_v, [vb, vnb, vrow, iota + k_full * _L], v,
                            mask=tail_mask)

                pltpu.async_copy(
                    pack_v.at[b],
                    out_hbm.at[pl.ds(base + bi0, nb)],
                    osems[b],
                )
            return carry

        lax.fori_loop(0, n_iter, outer, 0)
        for b in range(2):
            pltpu.make_async_copy(
                pack_v.at[b],
                out_hbm.at[pl.ds(0, nb)],
                osems[b],
            ).wait()

    return emb


def kernel(inputs, embs_weight):
    bsz, seq = inputs.shape
    n_vocab, d = embs_weight.shape
    dp = -(-d // _L) * _L
    tab_flat = jnp.pad(embs_weight, ((0, 0), (0, dp - d))).reshape(-1)
    return _build(bsz, seq, d, n_vocab)(inputs.reshape(-1), tab_flat)


# speedup vs baseline: 4.1921x; 1.6137x over previous
import functools

import jax
import jax.numpy as jnp
from jax import lax
from jax.experimental import pallas as pl
from jax.experimental.pallas import tpu as pltpu
from jax.experimental.pallas import tpu_sc as plsc

_N_SPECIAL = 8
_PAD_IDX = _N_SPECIAL

_NC = 2
_NS = 16
_NW = _NC * _NS
_L = 16


@functools.lru_cache(maxsize=None)
def _build(bsz: int, seq: int, d: int, n_vocab: int):
    dp = -(-d // _L) * _L
    assert bsz % (_NW * _L) == 0 and seq % 2 == 0
    b_per_w = bsz // _NW
    n_grp = b_per_w // _L
    mesh = plsc.VectorSubcoreMesh(core_axis_name="c", subcore_axis_name="s")

    @functools.partial(
        pl.kernel,
        mesh=mesh,
        out_type=jax.ShapeDtypeStruct((seq, d, bsz), jnp.float32),
        scratch_types=[
            pltpu.VMEM((b_per_w * seq,), jnp.int32),
            pltpu.VMEM((n_vocab * dp,), jnp.float32),
            pltpu.VMEM((2, dp, b_per_w), jnp.float32),
            pltpu.SemaphoreType.DMA,
            pltpu.SemaphoreType.DMA,
        ],
        compiler_params=pltpu.CompilerParams(
            use_tc_tiling_on_sc=True, needs_layout_passes=False),
    )
    def emb(idx_hbm, tab_hbm, out_hbm, idx_v, tab_v, pack_v, osem0, osem1):
        wid = lax.axis_index("s") * _NC + lax.axis_index("c")
        base = wid * b_per_w
        pltpu.sync_copy(idx_hbm.at[pl.ds(base * seq, b_per_w * seq)], idx_v)
        pltpu.sync_copy(tab_hbm, tab_v)

        zeros = jnp.zeros((_L,), jnp.float32)
        for r in (0, _PAD_IDX):
            for k in range(dp // _L):
                tab_v[pl.ds(r * dp + k * _L, _L)] = zeros

        iota = lax.iota(jnp.int32, _L)
        osems = (osem0, osem1)

        def outer(ll, carry):
            for b in range(2):
                l = ll * 2 + b

                @pl.when(ll > 0)
                def _drain():
                    pltpu.make_async_copy(
                        pack_v.at[b].at[pl.ds(0, d)],
                        out_hbm.at[0].at[:, pl.ds(0, b_per_w)],
                        osems[b],
                    ).wait()

                srcbs = []
                for g in range(n_grp):
                    vidx = plsc.load_gather(
                        idx_v, [(g * _L + iota) * seq + l])
                    srcbs.append(vidx * dp)

                @plsc.parallel_loop(0, d, unroll=2)
                def _col(c):
                    for g in range(n_grp):
                        v = plsc.load_gather(tab_v, [srcbs[g] + c])
                        pack_v[b, c, pl.ds(g * _L, _L)] = v

                pltpu.async_copy(
                    pack_v.at[b].at[pl.ds(0, d)],
                    out_hbm.at[l].at[:, pl.ds(base, b_per_w)],
                    osems[b],
                )
            return carry

        lax.fori_loop(0, seq // 2, outer, 0)
        for b in range(2):
            pltpu.make_async_copy(
                pack_v.at[b].at[pl.ds(0, d)],
                out_hbm.at[0].at[:, pl.ds(0, b_per_w)],
                osems[b],
            ).wait()

    return emb


def kernel(inputs, embs_weight):
    bsz, seq = inputs.shape
    n_vocab, d = embs_weight.shape
    dp = -(-d // _L) * _L
    tab_flat = jnp.pad(embs_weight, ((0, 0), (0, dp - d))).reshape(-1)
    out = _build(bsz, seq, d, n_vocab)(inputs.reshape(-1), tab_flat)
    return jnp.transpose(out, (2, 0, 1))


# speedup vs baseline: 4.1955x; 1.0008x over previous
import functools

import jax
import jax.numpy as jnp
from jax import lax
from jax.experimental import pallas as pl
from jax.experimental.pallas import tpu as pltpu
from jax.experimental.pallas import tpu_sc as plsc

_N_SPECIAL = 8
_PAD_IDX = _N_SPECIAL

_NC = 2
_NS = 16
_NW = _NC * _NS
_L = 16


@functools.lru_cache(maxsize=None)
def _build(bsz: int, seq: int, d: int, n_vocab: int):
    dp = -(-d // _L) * _L
    assert bsz % (_NW * _L) == 0 and seq % 2 == 0
    b_per_w = bsz // _NW
    n_grp = b_per_w // _L
    mesh = plsc.VectorSubcoreMesh(core_axis_name="c", subcore_axis_name="s")

    @functools.partial(
        pl.kernel,
        mesh=mesh,
        out_type=jax.ShapeDtypeStruct((seq, d, bsz), jnp.float32),
        scratch_types=[
            pltpu.VMEM((b_per_w * seq,), jnp.int32),
            pltpu.VMEM((n_vocab * dp,), jnp.float32),
            pltpu.VMEM((2, dp, b_per_w), jnp.float32),
            pltpu.SemaphoreType.DMA,
            pltpu.SemaphoreType.DMA,
        ],
        compiler_params=pltpu.CompilerParams(
            use_tc_tiling_on_sc=True, needs_layout_passes=False),
    )
    def emb(idx_hbm, tab_hbm, out_hbm, idx_v, tab_v, pack_v, osem0, osem1):
        wid = lax.axis_index("s") * _NC + lax.axis_index("c")
        base = wid * b_per_w
        pltpu.sync_copy(idx_hbm.at[pl.ds(base * seq, b_per_w * seq)], idx_v)
        pltpu.sync_copy(tab_hbm, tab_v)

        zeros = jnp.zeros((_L,), jnp.float32)
        for r in (0, _PAD_IDX):
            for k in range(dp // _L):
                tab_v[pl.ds(r * dp + k * _L, _L)] = zeros

        iota = lax.iota(jnp.int32, _L)
        osems = (osem0, osem1)

        def outer(ll, carry):
            for b in range(2):
                l = ll * 2 + b

                @pl.when(ll > 0)
                def _drain():
                    pltpu.make_async_copy(
                        pack_v.at[b].at[pl.ds(0, d)],
                        out_hbm.at[0].at[:, pl.ds(0, b_per_w)],
                        osems[b],
                    ).wait()

                srcbs = []
                for g in range(n_grp):
                    vidx = plsc.load_gather(
                        idx_v, [(g * _L + iota) * seq + l])
                    srcbs.append(vidx * dp)

                @plsc.parallel_loop(0, d, unroll=4)
                def _col(c):
                    for g in range(n_grp):
                        v = plsc.load_gather(tab_v, [srcbs[g] + c])
                        pack_v[b, c, pl.ds(g * _L, _L)] = v

                pltpu.async_copy(
                    pack_v.at[b].at[pl.ds(0, d)],
                    out_hbm.at[l].at[:, pl.ds(base, b_per_w)],
                    osems[b],
                )
            return carry

        lax.fori_loop(0, seq // 2, outer, 0)
        for b in range(2):
            pltpu.make_async_copy(
                pack_v.at[b].at[pl.ds(0, d)],
                out_hbm.at[0].at[:, pl.ds(0, b_per_w)],
                osems[b],
            ).wait()

    return emb


def kernel(inputs, embs_weight):
    bsz, seq = inputs.shape
    n_vocab, d = embs_weight.shape
    dp = -(-d // _L) * _L
    tab_flat = jnp.pad(embs_weight, ((0, 0), (0, dp - d))).reshape(-1)
    out = _build(bsz, seq, d, n_vocab)(inputs.reshape(-1), tab_flat)
    return jnp.transpose(out, (2, 0, 1))


# speedup vs baseline: 14.7848x; 3.5240x over previous
import functools

import jax
import jax.numpy as jnp
from jax import lax
from jax.experimental import pallas as pl
from jax.experimental.pallas import tpu as pltpu
from jax.experimental.pallas import tpu_sc as plsc

_N_SPECIAL = 8
_PAD_IDX = _N_SPECIAL

_NC = 2
_NS = 16
_NW = _NC * _NS
_L = 16


@functools.lru_cache(maxsize=None)
def _build(bsz: int, seq: int, d: int, n_vocab: int):
    dp = -(-d // _L) * _L + 1
    tab_words = -(-n_vocab * dp // _L) * _L
    assert bsz % (_NW * _L) == 0 and seq % 2 == 0
    b_per_w = bsz // _NW
    n_grp = b_per_w // _L
    mesh = plsc.VectorSubcoreMesh(core_axis_name="c", subcore_axis_name="s")

    @functools.partial(
        pl.kernel,
        mesh=mesh,
        out_type=jax.ShapeDtypeStruct((seq, d, bsz), jnp.float32),
        scratch_types=[
            pltpu.VMEM((b_per_w * seq,), jnp.int32),
            pltpu.VMEM((tab_words,), jnp.float32),
            pltpu.VMEM((2, -(-d // 8) * 8, b_per_w), jnp.float32),
            pltpu.SemaphoreType.DMA,
            pltpu.SemaphoreType.DMA,
        ],
        compiler_params=pltpu.CompilerParams(
            use_tc_tiling_on_sc=True, needs_layout_passes=False),
    )
    def emb(idx_hbm, tab_hbm, out_hbm, idx_v, tab_v, pack_v, osem0, osem1):
        wid = lax.axis_index("s") * _NC + lax.axis_index("c")
        base = wid * b_per_w
        pltpu.sync_copy(idx_hbm.at[pl.ds(base * seq, b_per_w * seq)], idx_v)
        pltpu.sync_copy(tab_hbm, tab_v)

        zeros = jnp.zeros((_L,), jnp.float32)
        iota = lax.iota(jnp.int32, _L)
        for r in (0, _PAD_IDX):
            for k in range(-(-d // _L)):
                plsc.store_scatter(
                    tab_v, [r * dp + k * _L + iota], zeros)
        osems = (osem0, osem1)

        def outer(ll, carry):
            for b in range(2):
                l = ll * 2 + b

                @pl.when(ll > 0)
                def _drain():
                    pltpu.make_async_copy(
                        pack_v.at[b].at[pl.ds(0, d)],
                        out_hbm.at[0].at[:, pl.ds(0, b_per_w)],
                        osems[b],
                    ).wait()

                srcbs = []
                for g in range(n_grp):
                    vidx = plsc.load_gather(
                        idx_v, [(g * _L + iota) * seq + l])
                    srcbs.append(vidx * dp)

                @plsc.parallel_loop(0, d, unroll=4)
                def _col(c):
                    for g in range(n_grp):
                        v = plsc.load_gather(tab_v, [srcbs[g] + c])
                        pack_v[b, c, pl.ds(g * _L, _L)] = v

                pltpu.async_copy(
                    pack_v.at[b].at[pl.ds(0, d)],
                    out_hbm.at[l].at[:, pl.ds(base, b_per_w)],
                    osems[b],
                )
            return carry

        lax.fori_loop(0, seq // 2, outer, 0)
        for b in range(2):
            pltpu.make_async_copy(
                pack_v.at[b].at[pl.ds(0, d)],
                out_hbm.at[0].at[:, pl.ds(0, b_per_w)],
                osems[b],
            ).wait()

    return emb


def kernel(inputs, embs_weight):
    bsz, seq = inputs.shape
    n_vocab, d = embs_weight.shape
    dp = -(-d // _L) * _L + 1
    tab_words = -(-n_vocab * dp // _L) * _L
    tab_flat = jnp.pad(embs_weight, ((0, 0), (0, dp - d))).reshape(-1)
    tab_flat = jnp.pad(tab_flat, (0, tab_words - n_vocab * dp))
    out = _build(bsz, seq, d, n_vocab)(inputs.reshape(-1), tab_flat)
    return jnp.transpose(out, (2, 0, 1))
